# baseline (speedup 1.0000x reference)
import functools

import jax
import jax.numpy as jnp
from jax import lax
from jax.experimental import pallas as pl
from jax.experimental.pallas import tpu as pltpu
from jax.experimental.pallas import tpu_sc as plsc

GRID_EDGE = 1024
NCELL = 2 * GRID_EDGE * GRID_EDGE
OB_LEN = 64
OUT_LEN = OB_LEN + NCELL
NB, NJ, NL = 1_000_000, 500_000, 100_000
NPTS = NB + 2 * NJ + NL
NW = 32
CHUNK = 4096
NSL = CHUNK // 128
NCH = ((NPTS + NW - 1) // NW + CHUNK - 1) // CHUNK
TPW = NCH * CHUNK
NPAD = NW * TPW
MAGIC = 12582912.0

_mesh = plsc.VectorSubcoreMesh(core_axis_name="c", subcore_axis_name="s")


@functools.partial(
    pl.kernel,
    mesh=_mesh,
    scratch_types=[
        pltpu.VMEM((3 * CHUNK,), jnp.int32),
        pltpu.VMEM((CHUNK,), jnp.int32),
        pltpu.VMEM((128,), jnp.float32),
        pltpu.VMEM((16,), jnp.float32),
        pltpu.VMEM((16,), jnp.float32),
        pltpu.SemaphoreType.DMA,
    ],
)
def _scatter_grid(packed, zx16, zy16, ones_hbm, out_ref,
                  xyz_v, idx_v, ones_v, zx_v, zy_v, sem):
    wid = lax.axis_index("s") * 2 + lax.axis_index("c")
    pltpu.sync_copy(zx16, zx_v)
    pltpu.sync_copy(zy16, zy_v)
    pltpu.sync_copy(ones_hbm, ones_v)
    zx = zx_v[...]
    zy = zy_v[...]

    def _to_grid(coord, zero):
        t = (coord - zero) * float(GRID_EDGE)
        t = jnp.minimum(jnp.maximum(t, -1.0), float(GRID_EDGE + 1))
        t = (t + MAGIC) - MAGIC
        t = jnp.minimum(jnp.maximum(t, 0.0), float(GRID_EDGE - 1))
        return t.astype(jnp.int32)

    def chunk_body(ci, carry):
        g = wid * NCH + ci
        pltpu.sync_copy(packed.at[pl.ds(g * (3 * CHUNK), 3 * CHUNK)], xyz_v)

        def vec_body(i, c):
            s = i * 16
            xv = lax.bitcast_convert_type(xyz_v[pl.ds(s, 16)], jnp.float32)
            yv = lax.bitcast_convert_type(
                xyz_v[pl.ds(CHUNK + s, 16)], jnp.float32)
            dv = xyz_v[pl.ds(2 * CHUNK + s, 16)]
            ix = _to_grid(xv, zx)
            iy = _to_grid(yv, zy)
            flat = ((dv << 20) | (ix << 10) | iy) + OB_LEN
            idx_v[pl.ds(s, 16)] = flat
            return c

        lax.fori_loop(0, CHUNK // 16, vec_body, 0)
        copies = [
            pltpu.async_copy(
                ones_v, out_ref.at[idx_v.at[pl.ds(j * 128, 128)]], sem)
            for j in range(NSL)
        ]
        for cp in copies:
            cp.wait()
        return carry

    lax.fori_loop(0, NCH, chunk_body, 0)


def kernel(ob, body_pos, body_depth, joint_posA, joint_posB, joint_depth,
           lidar_p2, hull):
    zero_x = hull[0] - 0.5
    zero_y = hull[1] - 0.5
    pad = NPAD - NPTS
    pos = jnp.concatenate(
        [body_pos, joint_posA, joint_posB, lidar_p2, body_pos[:pad]], axis=0)
    ds = jnp.concatenate(
        [body_depth, joint_depth, joint_depth,
         jnp.zeros((NL,), jnp.int32), body_depth[:pad]])
    xi = lax.bitcast_convert_type(pos[:, 0], jnp.int32).reshape(-1, CHUNK)
    yi = lax.bitcast_convert_type(pos[:, 1], jnp.int32).reshape(-1, CHUNK)
    di = ds.reshape(-1, CHUNK)
    packed = jnp.stack([xi, yi, di], axis=1).reshape(-1)
    zx16 = jnp.full((16,), zero_x, jnp.float32)
    zy16 = jnp.full((16,), zero_y, jnp.float32)
    ones = jnp.ones((128,), jnp.float32)
    init = jnp.concatenate(
        [ob.reshape(OB_LEN), jnp.zeros((NCELL,), jnp.float32)])
    buf = jax.new_ref(init)
    _scatter_grid(packed, zx16, zy16, ones, buf)
    return buf[...].reshape(1, OUT_LEN)


# speedup vs baseline: 5.1111x; 5.1111x over previous
import functools

import jax
import jax.numpy as jnp
from jax import lax
from jax.experimental import pallas as pl
from jax.experimental.pallas import tpu as pltpu
from jax.experimental.pallas import tpu_sc as plsc

GRID_EDGE = 1024
GRID1 = GRID_EDGE * GRID_EDGE
DUMPN = 2048
GRID_S = GRID1 + DUMPN
OB_LEN = 64
OUT_LEN = OB_LEN + 2 * GRID1
NB, NJ, NL = 1_000_000, 500_000, 100_000
NPTS = NB + 2 * NJ + NL
NSUB = 16
CHUNK = 2048
NSL = CHUNK // 128
NCH = ((NPTS + NSUB - 1) // NSUB + CHUNK - 1) // CHUNK
TPS = NCH * CHUNK
NPAD = NSUB * TPS
SLICE = GRID1 // NSUB
MAGIC = 12582912.0

_mesh = plsc.VectorSubcoreMesh(core_axis_name="c", subcore_axis_name="s")


@functools.partial(
    pl.kernel,
    mesh=_mesh,
    out_type=jax.ShapeDtypeStruct((OUT_LEN,), jnp.float32),
    scratch_types=[
        pltpu.VMEM_SHARED((GRID_S,), jnp.float32),
        pltpu.VMEM((3 * CHUNK,), jnp.int32),
        pltpu.VMEM((CHUNK,), jnp.int32),
        pltpu.VMEM((128,), jnp.float32),
        pltpu.VMEM((64,), jnp.float32),
        pltpu.VMEM((16384,), jnp.float32),
        pltpu.VMEM((16,), jnp.float32),
        pltpu.VMEM((16,), jnp.float32),
    ],
)
def _scatter_grid(packed, ob_in, zeros_hbm, zx16, zy16, ones_hbm, out_ref,
                  grid_s, xyz_v, idx_v, ones_v, ob_v, bounce_v, zx_v, zy_v):
    cid = lax.axis_index("c")
    sid = lax.axis_index("s")
    pltpu.sync_copy(zx16, zx_v)
    pltpu.sync_copy(zy16, zy_v)
    pltpu.sync_copy(ones_hbm, ones_v)
    zx = zx_v[...]
    zy = zy_v[...]

    pltpu.sync_copy(zeros_hbm, bounce_v)
    for k in range(SLICE // 16384):
        pltpu.sync_copy(bounce_v,
                        grid_s.at[pl.ds(sid * SLICE + k * 16384, 16384)])
    plsc.subcore_barrier()

    def _to_grid(coord, zero):
        t = (coord - zero) * float(GRID_EDGE)
        t = jnp.minimum(jnp.maximum(t, -1.0), float(GRID_EDGE + 1))
        t = (t + MAGIC) - MAGIC
        t = jnp.minimum(jnp.maximum(t, 0.0), float(GRID_EDGE - 1))
        return t.astype(jnp.int32)

    def chunk_body(ci, carry):
        g = sid * NCH + ci
        pltpu.sync_copy(packed.at[pl.ds(g * (3 * CHUNK), 3 * CHUNK)], xyz_v)

        def vec_body(i, c):
            s = i * 16
            xv = lax.bitcast_convert_type(xyz_v[pl.ds(s, 16)], jnp.float32)
            yv = lax.bitcast_convert_type(
                xyz_v[pl.ds(CHUNK + s, 16)], jnp.float32)
            dv = xyz_v[pl.ds(2 * CHUNK + s, 16)]
            ix = _to_grid(xv, zx)
            iy = _to_grid(yv, zy)
            flat = (ix << 10) | iy
            flat = jnp.where(dv == cid, flat, GRID1 + (flat & (DUMPN - 1)))
            idx_v[pl.ds(s, 16)] = flat
            return c

        lax.fori_loop(0, CHUNK // 16, vec_body, 0)
        for j in range(NSL):
            pltpu.sync_copy(ones_v, grid_s.at[idx_v.at[pl.ds(j * 128, 128)]])
        return carry

    lax.fori_loop(0, NCH, chunk_body, 0)
    plsc.subcore_barrier()

    for k in range(SLICE // 16384):
        pltpu.sync_copy(grid_s.at[pl.ds(sid * SLICE + k * 16384, 16384)],
                        bounce_v)
        pltpu.sync_copy(
            bounce_v,
            out_ref.at[pl.ds(OB_LEN + cid * GRID1 + sid * SLICE + k * 16384,
                             16384)])

    @pl.when(jnp.logical_and(cid == 0, sid == 0))
    def _():
        pltpu.sync_copy(ob_in, ob_v)
        pltpu.sync_copy(ob_v, out_ref.at[pl.ds(0, OB_LEN)])


def kernel(ob, body_pos, body_depth, joint_posA, joint_posB, joint_depth,
           lidar_p2, hull):
    zero_x = hull[0] - 0.5
    zero_y = hull[1] - 0.5
    pad = NPAD - NPTS
    pos = jnp.concatenate(
        [body_pos, joint_posA, joint_posB, lidar_p2, body_pos[:pad]], axis=0)
    ds = jnp.concatenate(
        [body_depth, joint_depth, joint_depth,
         jnp.zeros((NL,), jnp.int32), body_depth[:pad]])
    xi = lax.bitcast_convert_type(pos[:, 0], jnp.int32).reshape(-1, CHUNK)
    yi = lax.bitcast_convert_type(pos[:, 1], jnp.int32).reshape(-1, CHUNK)
    di = ds.reshape(-1, CHUNK)
    packed = jnp.stack([xi, yi, di], axis=1).reshape(-1)
    zx16 = jnp.full((16,), zero_x, jnp.float32)
    zy16 = jnp.full((16,), zero_y, jnp.float32)
    ones = jnp.ones((128,), jnp.float32)
    zeros = jnp.zeros((16384,), jnp.float32)
    out = _scatter_grid(packed, ob.reshape(OB_LEN), zeros, zx16, zy16, ones)
    return out.reshape(1, OUT_LEN)


# speedup vs baseline: 5.1151x; 1.0008x over previous
import functools

import jax
import jax.numpy as jnp
from jax import lax
from jax.experimental import pallas as pl
from jax.experimental.pallas import tpu as pltpu
from jax.experimental.pallas import tpu_sc as plsc

GRID_EDGE = 1024
GRID1 = GRID_EDGE * GRID_EDGE
DUMPN = 2048
GRID_S = GRID1 + DUMPN
OB_LEN = 64
OUT_LEN = OB_LEN + 2 * GRID1
NB, NJ, NL = 1_000_000, 500_000, 100_000
NPTS = NB + 2 * NJ + NL
NSUB = 16
CHUNK = 2048
NSL = CHUNK // 128
NCH = ((NPTS + NSUB - 1) // NSUB + CHUNK - 1) // CHUNK
TPS = NCH * CHUNK
NPAD = NSUB * TPS
SLICE = GRID1 // NSUB
MAGIC = 12582912.0

_mesh = plsc.VectorSubcoreMesh(core_axis_name="c", subcore_axis_name="s")


@functools.partial(
    pl.kernel,
    mesh=_mesh,
    out_type=jax.ShapeDtypeStruct((OUT_LEN,), jnp.float32),
    scratch_types=[
        pltpu.VMEM_SHARED((GRID_S,), jnp.float32),
        pltpu.VMEM((3 * CHUNK,), jnp.int32),
        pltpu.VMEM((CHUNK,), jnp.int32),
        pltpu.VMEM((CHUNK,), jnp.float32),
        pltpu.VMEM((64,), jnp.float32),
        pltpu.VMEM((16384,), jnp.float32),
        pltpu.VMEM((16,), jnp.float32),
        pltpu.VMEM((16,), jnp.float32),
    ],
)
def _scatter_grid(packed, ob_in, zeros_hbm, zx16, zy16, ones_hbm, out_ref,
                  grid_s, xyz_v, idx_v, ones_v, ob_v, bounce_v, zx_v, zy_v):
    cid = lax.axis_index("c")
    sid = lax.axis_index("s")
    pltpu.sync_copy(zx16, zx_v)
    pltpu.sync_copy(zy16, zy_v)
    pltpu.sync_copy(ones_hbm, ones_v)
    zx = zx_v[...]
    zy = zy_v[...]

    pltpu.sync_copy(zeros_hbm, bounce_v)
    for k in range(SLICE // 16384):
        pltpu.sync_copy(bounce_v,
                        grid_s.at[pl.ds(sid * SLICE + k * 16384, 16384)])
    plsc.subcore_barrier()

    def _to_grid(coord, zero):
        t = (coord - zero) * float(GRID_EDGE)
        t = jnp.minimum(jnp.maximum(t, -1.0), float(GRID_EDGE + 1))
        t = (t + MAGIC) - MAGIC
        t = jnp.minimum(jnp.maximum(t, 0.0), float(GRID_EDGE - 1))
        return t.astype(jnp.int32)

    def chunk_body(ci, carry):
        g = sid * NCH + ci
        pltpu.sync_copy(packed.at[pl.ds(g * (3 * CHUNK), 3 * CHUNK)], xyz_v)

        def vec_body(i, c):
            s = i * 16
            xv = lax.bitcast_convert_type(xyz_v[pl.ds(s, 16)], jnp.float32)
            yv = lax.bitcast_convert_type(
                xyz_v[pl.ds(CHUNK + s, 16)], jnp.float32)
            dv = xyz_v[pl.ds(2 * CHUNK + s, 16)]
            ix = _to_grid(xv, zx)
            iy = _to_grid(yv, zy)
            flat = (ix << 10) | iy
            flat = jnp.where(dv == cid, flat, GRID1 + (flat & (DUMPN - 1)))
            idx_v[pl.ds(s, 16)] = flat
            return c

        lax.fori_loop(0, CHUNK // 16, vec_body, 0)
        pltpu.sync_copy(ones_v, grid_s.at[idx_v])
        return carry

    lax.fori_loop(0, NCH, chunk_body, 0)
    plsc.subcore_barrier()

    for k in range(SLICE // 16384):
        pltpu.sync_copy(grid_s.at[pl.ds(sid * SLICE + k * 16384, 16384)],
                        bounce_v)
        pltpu.sync_copy(
            bounce_v,
            out_ref.at[pl.ds(OB_LEN + cid * GRID1 + sid * SLICE + k * 16384,
                             16384)])

    @pl.when(jnp.logical_and(cid == 0, sid == 0))
    def _():
        pltpu.sync_copy(ob_in, ob_v)
        pltpu.sync_copy(ob_v, out_ref.at[pl.ds(0, OB_LEN)])


def kernel(ob, body_pos, body_depth, joint_posA, joint_posB, joint_depth,
           lidar_p2, hull):
    zero_x = hull[0] - 0.5
    zero_y = hull[1] - 0.5
    pad = NPAD - NPTS
    pos = jnp.concatenate(
        [body_pos, joint_posA, joint_posB, lidar_p2, body_pos[:pad]], axis=0)
    ds = jnp.concatenate(
        [body_depth, joint_depth, joint_depth,
         jnp.zeros((NL,), jnp.int32), body_depth[:pad]])
    xi = lax.bitcast_convert_type(pos[:, 0], jnp.int32).reshape(-1, CHUNK)
    yi = lax.bitcast_convert_type(pos[:, 1], jnp.int32).reshape(-1, CHUNK)
    di = ds.reshape(-1, CHUNK)
    packed = jnp.stack([xi, yi, di], axis=1).reshape(-1)
    zx16 = jnp.full((16,), zero_x, jnp.float32)
    zy16 = jnp.full((16,), zero_y, jnp.float32)
    ones = jnp.ones((CHUNK,), jnp.float32)
    zeros = jnp.zeros((16384,), jnp.float32)
    out = _scatter_grid(packed, ob.reshape(OB_LEN), zeros, zx16, zy16, ones)
    return out.reshape(1, OUT_LEN)


# speedup vs baseline: 7.1638x; 1.4005x over previous
import functools

import jax
import jax.numpy as jnp
from jax import lax
from jax.experimental import pallas as pl
from jax.experimental.pallas import tpu as pltpu
from jax.experimental.pallas import tpu_sc as plsc

GRID_EDGE = 1024
GRID1 = GRID_EDGE * GRID_EDGE
DUMPN = 2048
GRID_S = GRID1 + DUMPN
OB_LEN = 64
OUT_LEN = OB_LEN + 2 * GRID1
NB, NJ, NL = 1_000_000, 500_000, 100_000
NSUB = 16
CHUNK = 2048
SLICE = GRID1 // NSUB
WB = 16384
MAGIC = 12582912.0

_mesh = plsc.VectorSubcoreMesh(core_axis_name="c", subcore_axis_name="s")


@functools.partial(
    pl.kernel,
    mesh=_mesh,
    out_type=jax.ShapeDtypeStruct((OUT_LEN,), jnp.float32),
    scratch_types=[
        pltpu.VMEM_SHARED((GRID_S,), jnp.float32),
        pltpu.VMEM((CHUNK,), jnp.float32),
        pltpu.VMEM((CHUNK,), jnp.float32),
        pltpu.VMEM((CHUNK,), jnp.int32),
        pltpu.VMEM((CHUNK,), jnp.int32),
        pltpu.VMEM((CHUNK,), jnp.float32),
        pltpu.VMEM((64,), jnp.float32),
        pltpu.VMEM((WB,), jnp.float32),
        pltpu.VMEM((16,), jnp.float32),
        pltpu.VMEM((16,), jnp.float32),
    ],
)
def _scatter_grid(body_xy, body_d, jA_xy, jB_xy, joint_d, lidar_xy,
                  ob_in, zeros_hbm, zx16, zy16, ones_hbm, out_ref,
                  grid_s, x_v, y_v, d_v, idx_v, ones_v, ob_v, bounce_v,
                  zx_v, zy_v):
    cid = lax.axis_index("c")
    sid = lax.axis_index("s")
    pltpu.sync_copy(zx16, zx_v)
    pltpu.sync_copy(zy16, zy_v)
    pltpu.sync_copy(ones_hbm, ones_v)

    pltpu.sync_copy(zeros_hbm, bounce_v)
    for k in range(SLICE // WB):
        pltpu.sync_copy(bounce_v,
                        grid_s.at[pl.ds(sid * SLICE + k * WB, WB)])
    plsc.subcore_barrier()

    def _to_grid(coord, zero):
        t = (coord - zero) * float(GRID_EDGE)
        t = jnp.minimum(jnp.maximum(t, -1.0), float(GRID_EDGE + 1))
        t = (t + MAGIC) - MAGIC
        t = jnp.minimum(jnp.maximum(t, 0.0), float(GRID_EDGE - 1))
        return t.astype(jnp.int32)

    def _phase(xy_ref, d_ref, npts):
        nch = (npts + CHUNK - 1) // CHUNK
        nloop = (nch + NSUB - 1) // NSUB

        def chunk_body(k, carry):
            gi = k * NSUB + sid
            off = jnp.minimum(gi * CHUNK, npts - CHUNK)
            pltpu.sync_copy(xy_ref.at[pl.ds(off, CHUNK)], x_v)
            pltpu.sync_copy(xy_ref.at[pl.ds(npts + off, CHUNK)], y_v)
            if d_ref is not None:
                pltpu.sync_copy(d_ref.at[pl.ds(off, CHUNK)], d_v)

            def vec_body(i, c):
                s = i * 16
                ix = _to_grid(x_v[pl.ds(s, 16)], zx_v[...])
                iy = _to_grid(y_v[pl.ds(s, 16)], zy_v[...])
                flat = (ix << 10) | iy
                if d_ref is not None:
                    dv = d_v[pl.ds(s, 16)]
                    flat = jnp.where(dv == cid, flat,
                                     GRID1 + (flat & (DUMPN - 1)))
                idx_v[pl.ds(s, 16)] = flat
                return c

            lax.fori_loop(0, CHUNK // 16, vec_body, 0)
            pltpu.sync_copy(ones_v, grid_s.at[idx_v])
            return carry

        lax.fori_loop(0, nloop, chunk_body, 0)

    _phase(body_xy, body_d, NB)
    _phase(jA_xy, joint_d, NJ)
    _phase(jB_xy, joint_d, NJ)

    @pl.when(cid == 0)
    def _():
        _phase(lidar_xy, None, NL)

    plsc.subcore_barrier()

    for k in range(SLICE // WB):
        pltpu.sync_copy(grid_s.at[pl.ds(sid * SLICE + k * WB, WB)],
                        bounce_v)
        pltpu.sync_copy(
            bounce_v,
            out_ref.at[pl.ds(OB_LEN + cid * GRID1 + sid * SLICE + k * WB,
                             WB)])

    @pl.when(jnp.logical_and(cid == 0, sid == 0))
    def _():
        pltpu.sync_copy(ob_in, ob_v)
        pltpu.sync_copy(ob_v, out_ref.at[pl.ds(0, OB_LEN)])


def kernel(ob, body_pos, body_depth, joint_posA, joint_posB, joint_depth,
           lidar_p2, hull):
    zx16 = jnp.full((16,), hull[0] - 0.5, jnp.float32)
    zy16 = jnp.full((16,), hull[1] - 0.5, jnp.float32)
    ones = jnp.ones((CHUNK,), jnp.float32)
    zeros = jnp.zeros((WB,), jnp.float32)
    out = _scatter_grid(
        body_pos.T.reshape(2 * NB), body_depth,
        joint_posA.T.reshape(2 * NJ), joint_posB.T.reshape(2 * NJ),
        joint_depth, lidar_p2.T.reshape(2 * NL),
        ob.reshape(OB_LEN), zeros, zx16, zy16, ones)
    return out.reshape(1, OUT_LEN)


# speedup vs baseline: 7.1679x; 1.0006x over previous
import functools

import jax
import jax.numpy as jnp
from jax import lax
from jax.experimental import pallas as pl
from jax.experimental.pallas import tpu as pltpu
from jax.experimental.pallas import tpu_sc as plsc

GRID_EDGE = 1024
GRID1 = GRID_EDGE * GRID_EDGE
DUMPN = 2048
GRID_S = GRID1 + DUMPN
OB_LEN = 64
OUT_LEN = OB_LEN + 2 * GRID1
NB, NJ, NL = 1_000_000, 500_000, 100_000
NSUB = 16
CHUNK = 2048
SLICE = GRID1 // NSUB
WB = 16384
MAGIC = 12582912.0

_mesh = plsc.VectorSubcoreMesh(core_axis_name="c", subcore_axis_name="s")


@functools.partial(
    pl.kernel,
    mesh=_mesh,
    out_type=jax.ShapeDtypeStruct((OUT_LEN,), jnp.float32),
    scratch_types=[
        pltpu.VMEM_SHARED((GRID_S,), jnp.float32),
        pltpu.VMEM((CHUNK,), jnp.float32),
        pltpu.VMEM((CHUNK,), jnp.float32),
        pltpu.VMEM((CHUNK,), jnp.int32),
        pltpu.VMEM((CHUNK,), jnp.float32),
        pltpu.VMEM((CHUNK,), jnp.float32),
        pltpu.VMEM((CHUNK,), jnp.int32),
        pltpu.SemaphoreType.DMA,
        pltpu.SemaphoreType.DMA,
        pltpu.VMEM((CHUNK,), jnp.int32),
        pltpu.VMEM((CHUNK,), jnp.float32),
        pltpu.VMEM((64,), jnp.float32),
        pltpu.VMEM((WB,), jnp.float32),
        pltpu.VMEM((16,), jnp.float32),
        pltpu.VMEM((16,), jnp.float32),
    ],
)
def _scatter_grid(body_xy, body_d, jA_xy, jB_xy, joint_d, lidar_xy,
                  ob_in, zeros_hbm, zx16, zy16, ones_hbm, out_ref,
                  grid_s, x_v, y_v, d_v, x2_v, y2_v, d2_v, sem0, sem1,
                  idx_v, ones_v, ob_v, bounce_v, zx_v, zy_v):
    cid = lax.axis_index("c")
    sid = lax.axis_index("s")
    pltpu.sync_copy(zx16, zx_v)
    pltpu.sync_copy(zy16, zy_v)
    pltpu.sync_copy(ones_hbm, ones_v)

    pltpu.sync_copy(zeros_hbm, bounce_v)
    for k in range(SLICE // WB):
        pltpu.sync_copy(bounce_v,
                        grid_s.at[pl.ds(sid * SLICE + k * WB, WB)])
    plsc.subcore_barrier()

    def _to_grid(coord, zero):
        t = (coord - zero) * float(GRID_EDGE)
        t = jnp.minimum(jnp.maximum(t, -1.0), float(GRID_EDGE + 1))
        t = (t + MAGIC) - MAGIC
        t = jnp.minimum(jnp.maximum(t, 0.0), float(GRID_EDGE - 1))
        return t.astype(jnp.int32)

    bufs = ((x_v, y_v, d_v, sem0), (x2_v, y2_v, d2_v, sem1))

    def _phase(xy_ref, d_ref, npts):
        nch = (npts + CHUNK - 1) // CHUNK
        nloop = (nch + NSUB - 1) // NSUB

        def _off(j):
            return jnp.minimum((j * NSUB + sid) * CHUNK, npts - CHUNK)

        def _start(j, b):
            xb, yb, db, sem = bufs[b]
            off = _off(j)
            pltpu.async_copy(xy_ref.at[pl.ds(off, CHUNK)], xb, sem)
            pltpu.async_copy(xy_ref.at[pl.ds(npts + off, CHUNK)], yb, sem)
            if d_ref is not None:
                pltpu.async_copy(d_ref.at[pl.ds(off, CHUNK)], db, sem)

        def _drain(b):
            xb, yb, db, sem = bufs[b]
            pltpu.make_async_copy(xy_ref.at[pl.ds(0, CHUNK)], xb, sem).wait()
            pltpu.make_async_copy(xy_ref.at[pl.ds(0, CHUNK)], yb, sem).wait()
            if d_ref is not None:
                pltpu.make_async_copy(d_ref.at[pl.ds(0, CHUNK)], db,
                                      sem).wait()

        _start(0, 0)
        for j in range(nloop):
            b = j % 2
            xb, yb, db, _ = bufs[b]
            _drain(b)
            if j + 1 < nloop:
                _start(j + 1, 1 - b)

            def vec_body(i, c, xb=xb, yb=yb, db=db):
                s = i * 16
                ix = _to_grid(xb[pl.ds(s, 16)], zx_v[...])
                iy = _to_grid(yb[pl.ds(s, 16)], zy_v[...])
                flat = (ix << 10) | iy
                if d_ref is not None:
                    dv = db[pl.ds(s, 16)]
                    flat = jnp.where(dv == cid, flat,
                                     GRID1 + (flat & (DUMPN - 1)))
                idx_v[pl.ds(s, 16)] = flat
                return c

            lax.fori_loop(0, CHUNK // 16, vec_body, 0)
            pltpu.sync_copy(ones_v, grid_s.at[idx_v])

    _phase(body_xy, body_d, NB)
    _phase(jA_xy, joint_d, NJ)
    _phase(jB_xy, joint_d, NJ)

    @pl.when(cid == 0)
    def _():
        _phase(lidar_xy, None, NL)

    plsc.subcore_barrier()

    for k in range(SLICE // WB):
        pltpu.sync_copy(grid_s.at[pl.ds(sid * SLICE + k * WB, WB)],
                        bounce_v)
        pltpu.sync_copy(
            bounce_v,
            out_ref.at[pl.ds(OB_LEN + cid * GRID1 + sid * SLICE + k * WB,
                             WB)])

    @pl.when(jnp.logical_and(cid == 0, sid == 0))
    def _():
        pltpu.sync_copy(ob_in, ob_v)
        pltpu.sync_copy(ob_v, out_ref.at[pl.ds(0, OB_LEN)])


def kernel(ob, body_pos, body_depth, joint_posA, joint_posB, joint_depth,
           lidar_p2, hull):
    zx16 = jnp.full((16,), hull[0] - 0.5, jnp.float32)
    zy16 = jnp.full((16,), hull[1] - 0.5, jnp.float32)
    ones = jnp.ones((CHUNK,), jnp.float32)
    zeros = jnp.zeros((WB,), jnp.float32)
    out = _scatter_grid(
        body_pos.T.reshape(2 * NB), body_depth,
        joint_posA.T.reshape(2 * NJ), joint_posB.T.reshape(2 * NJ),
        joint_depth, lidar_p2.T.reshape(2 * NL),
        ob.reshape(OB_LEN), zeros, zx16, zy16, ones)
    return out.reshape(1, OUT_LEN)


# speedup vs baseline: 7.1753x; 1.0010x over previous
import functools

import jax
import jax.numpy as jnp
from jax import lax
from jax.experimental import pallas as pl
from jax.experimental.pallas import tpu as pltpu
from jax.experimental.pallas import tpu_sc as plsc

GRID_EDGE = 1024
GRID1 = GRID_EDGE * GRID_EDGE
DUMPN = 2048
GRID_S = GRID1 + DUMPN
OB_LEN = 64
OUT_LEN = OB_LEN + 2 * GRID1
NB, NJ, NL = 1_000_000, 500_000, 100_000
NSUB = 16
CHUNK = 2048
SLICE = GRID1 // NSUB
WB = 16384
MAGIC = 12582912.0

_mesh = plsc.VectorSubcoreMesh(core_axis_name="c", subcore_axis_name="s")


@functools.partial(
    pl.kernel,
    mesh=_mesh,
    out_type=jax.ShapeDtypeStruct((OUT_LEN,), jnp.float32),
    scratch_types=[
        pltpu.VMEM_SHARED((GRID_S,), jnp.float32),
        pltpu.VMEM((CHUNK,), jnp.float32),
        pltpu.VMEM((CHUNK,), jnp.float32),
        pltpu.VMEM((CHUNK,), jnp.int32),
        pltpu.VMEM((CHUNK,), jnp.float32),
        pltpu.VMEM((CHUNK,), jnp.float32),
        pltpu.VMEM((CHUNK,), jnp.int32),
        pltpu.SemaphoreType.DMA,
        pltpu.SemaphoreType.DMA,
        pltpu.VMEM((CHUNK,), jnp.int32),
        pltpu.SemaphoreType.DMA,
        pltpu.SemaphoreType.DMA,
        pltpu.VMEM((CHUNK,), jnp.int32),
        pltpu.VMEM((CHUNK,), jnp.float32),
        pltpu.VMEM((64,), jnp.float32),
        pltpu.VMEM((WB,), jnp.float32),
        pltpu.VMEM((16,), jnp.float32),
        pltpu.VMEM((16,), jnp.float32),
    ],
)
def _scatter_grid(body_xy, body_d, jA_xy, jB_xy, joint_d, lidar_xy,
                  ob_in, zeros_hbm, zx16, zy16, ones_hbm, out_ref,
                  grid_s, x_v, y_v, d_v, x2_v, y2_v, d2_v, sem0, sem1,
                  idx2_v, ssem0, ssem1, idx_v, ones_v, ob_v, bounce_v,
                  zx_v, zy_v):
    cid = lax.axis_index("c")
    sid = lax.axis_index("s")
    pltpu.sync_copy(zx16, zx_v)
    pltpu.sync_copy(zy16, zy_v)
    pltpu.sync_copy(ones_hbm, ones_v)

    pltpu.sync_copy(zeros_hbm, bounce_v)
    for k in range(SLICE // WB):
        pltpu.sync_copy(bounce_v,
                        grid_s.at[pl.ds(sid * SLICE + k * WB, WB)])
    plsc.subcore_barrier()

    def _to_grid(coord, zero):
        t = (coord - zero) * float(GRID_EDGE)
        t = jnp.minimum(jnp.maximum(t, -1.0), float(GRID_EDGE + 1))
        t = (t + MAGIC) - MAGIC
        t = jnp.minimum(jnp.maximum(t, 0.0), float(GRID_EDGE - 1))
        return t.astype(jnp.int32)

    bufs = ((x_v, y_v, d_v, sem0), (x2_v, y2_v, d2_v, sem1))
    sbufs = ((idx_v, ssem0), (idx2_v, ssem1))
    pending = [False, False]

    def _drain_scatter(b):
        if pending[b]:
            ib, sem = sbufs[b]
            pltpu.make_async_copy(ones_v, grid_s.at[ib], sem).wait()
            pending[b] = False

    def _phase(xy_ref, d_ref, npts):
        nch = (npts + CHUNK - 1) // CHUNK
        nloop = (nch + NSUB - 1) // NSUB

        def _off(j):
            return jnp.minimum((j * NSUB + sid) * CHUNK, npts - CHUNK)

        def _start(j, b):
            xb, yb, db, sem = bufs[b]
            off = _off(j)
            pltpu.async_copy(xy_ref.at[pl.ds(off, CHUNK)], xb, sem)
            pltpu.async_copy(xy_ref.at[pl.ds(npts + off, CHUNK)], yb, sem)
            if d_ref is not None:
                pltpu.async_copy(d_ref.at[pl.ds(off, CHUNK)], db, sem)

        def _drain(b):
            xb, yb, db, sem = bufs[b]
            pltpu.make_async_copy(xy_ref.at[pl.ds(0, CHUNK)], xb, sem).wait()
            pltpu.make_async_copy(xy_ref.at[pl.ds(0, CHUNK)], yb, sem).wait()
            if d_ref is not None:
                pltpu.make_async_copy(d_ref.at[pl.ds(0, CHUNK)], db,
                                      sem).wait()

        _start(0, 0)
        for j in range(nloop):
            b = j % 2
            xb, yb, db, _ = bufs[b]
            ib, ssem = sbufs[b]
            _drain(b)
            if j + 1 < nloop:
                _start(j + 1, 1 - b)
            _drain_scatter(b)

            def vec_body(i, c, xb=xb, yb=yb, db=db, ib=ib):
                s = i * 16
                ix = _to_grid(xb[pl.ds(s, 16)], zx_v[...])
                iy = _to_grid(yb[pl.ds(s, 16)], zy_v[...])
                flat = (ix << 10) | iy
                if d_ref is not None:
                    dv = db[pl.ds(s, 16)]
                    flat = jnp.where(dv == cid, flat,
                                     GRID1 + (flat & (DUMPN - 1)))
                ib[pl.ds(s, 16)] = flat
                return c

            lax.fori_loop(0, CHUNK // 16, vec_body, 0)
            pltpu.async_copy(ones_v, grid_s.at[ib], ssem)
            pending[b] = True

    _phase(body_xy, body_d, NB)
    _phase(jA_xy, joint_d, NJ)
    _phase(jB_xy, joint_d, NJ)

    @pl.when(cid == 0)
    def _():
        _phase(lidar_xy, None, NL)

    _drain_scatter(0)
    _drain_scatter(1)
    plsc.subcore_barrier()

    for k in range(SLICE // WB):
        pltpu.sync_copy(grid_s.at[pl.ds(sid * SLICE + k * WB, WB)],
                        bounce_v)
        pltpu.sync_copy(
            bounce_v,
            out_ref.at[pl.ds(OB_LEN + cid * GRID1 + sid * SLICE + k * WB,
                             WB)])

    @pl.when(jnp.logical_and(cid == 0, sid == 0))
    def _():
        pltpu.sync_copy(ob_in, ob_v)
        pltpu.sync_copy(ob_v, out_ref.at[pl.ds(0, OB_LEN)])


def kernel(ob, body_pos, body_depth, joint_posA, joint_posB, joint_depth,
           lidar_p2, hull):
    zx16 = jnp.full((16,), hull[0] - 0.5, jnp.float32)
    zy16 = jnp.full((16,), hull[1] - 0.5, jnp.float32)
    ones = jnp.ones((CHUNK,), jnp.float32)
    zeros = jnp.zeros((WB,), jnp.float32)
    out = _scatter_grid(
        body_pos.T.reshape(2 * NB), body_depth,
        joint_posA.T.reshape(2 * NJ), joint_posB.T.reshape(2 * NJ),
        joint_depth, lidar_p2.T.reshape(2 * NL),
        ob.reshape(OB_LEN), zeros, zx16, zy16, ones)
    return out.reshape(1, OUT_LEN)


# speedup vs baseline: 7.2158x; 1.0056x over previous
import functools

import jax
import jax.numpy as jnp
from jax import lax
from jax.experimental import pallas as pl
from jax.experimental.pallas import tpu as pltpu
from jax.experimental.pallas import tpu_sc as plsc

GRID_EDGE = 1024
GRID1 = GRID_EDGE * GRID_EDGE
DUMPN = 2048
GRID_S = GRID1 + DUMPN
OB_LEN = 64
OUT_LEN = OB_LEN + 2 * GRID1
NB, NJ, NL = 1_000_000, 500_000, 100_000
NSUB = 16
CHUNK = 2048
SLICE = GRID1 // NSUB
WB = 16384
MAGIC = 12582912.0

_mesh = plsc.VectorSubcoreMesh(core_axis_name="c", subcore_axis_name="s")


@functools.partial(
    pl.kernel,
    mesh=_mesh,
    out_type=jax.ShapeDtypeStruct((OUT_LEN,), jnp.float32),
    scratch_types=[
        pltpu.VMEM_SHARED((GRID_S,), jnp.float32),
        pltpu.VMEM((CHUNK,), jnp.float32),
        pltpu.VMEM((CHUNK,), jnp.float32),
        pltpu.VMEM((CHUNK,), jnp.int32),
        pltpu.VMEM((CHUNK,), jnp.int32),
        pltpu.VMEM((CHUNK,), jnp.float32),
        pltpu.VMEM((64,), jnp.float32),
        pltpu.VMEM((WB,), jnp.float32),
        pltpu.VMEM((16,), jnp.float32),
        pltpu.VMEM((16,), jnp.float32),
    ],
)
def _scatter_grid(body_xy, body_d, jA_xy, jB_xy, joint_d, lidar_xy,
                  ob_in, zeros_hbm, zx16, zy16, ones_hbm, out_ref,
                  grid_s, x_v, y_v, d_v, idx_v, ones_v, ob_v, bounce_v,
                  zx_v, zy_v):
    cid = lax.axis_index("c")
    sid = lax.axis_index("s")
    pltpu.sync_copy(zx16, zx_v)
    pltpu.sync_copy(zy16, zy_v)
    pltpu.sync_copy(ones_hbm, ones_v)

    pltpu.sync_copy(zeros_hbm, bounce_v)
    for k in range(SLICE // WB):
        pltpu.sync_copy(bounce_v,
                        grid_s.at[pl.ds(sid * SLICE + k * WB, WB)])
    plsc.subcore_barrier()

    def _to_grid(coord, zero):
        t = (coord - zero) * float(GRID_EDGE)
        t = jnp.minimum(jnp.maximum(t, -1.0), float(GRID_EDGE + 1))
        t = (t + MAGIC) - MAGIC
        t = jnp.minimum(jnp.maximum(t, 0.0), float(GRID_EDGE - 1))
        return t.astype(jnp.int32)

    zx = zx_v[...]
    zy = zy_v[...]

    def _phase(xy_ref, d_ref, npts):
        nch = (npts + CHUNK - 1) // CHUNK
        nloop = (nch + NSUB - 1) // NSUB

        def chunk_body(k, carry):
            off = jnp.minimum((k * NSUB + sid) * CHUNK, npts - CHUNK)
            pltpu.sync_copy(xy_ref.at[pl.ds(off, CHUNK)], x_v)
            pltpu.sync_copy(xy_ref.at[pl.ds(npts + off, CHUNK)], y_v)
            if d_ref is not None:
                pltpu.sync_copy(d_ref.at[pl.ds(off, CHUNK)], d_v)

            def vec_body(i, c):
                s = i * 64
                for u in range(4):
                    su = s + u * 16
                    ix = _to_grid(x_v[pl.ds(su, 16)], zx)
                    iy = _to_grid(y_v[pl.ds(su, 16)], zy)
                    flat = (ix << 10) | iy
                    if d_ref is not None:
                        dv = d_v[pl.ds(su, 16)]
                        flat = jnp.where(dv == cid, flat,
                                         GRID1 + (flat & (DUMPN - 1)))
                    idx_v[pl.ds(su, 16)] = flat
                return c

            lax.fori_loop(0, CHUNK // 64, vec_body, 0)
            pltpu.sync_copy(ones_v, grid_s.at[idx_v])
            return carry

        lax.fori_loop(0, nloop, chunk_body, 0)

    _phase(body_xy, body_d, NB)
    _phase(jA_xy, joint_d, NJ)
    _phase(jB_xy, joint_d, NJ)

    @pl.when(cid == 0)
    def _():
        _phase(lidar_xy, None, NL)

    plsc.subcore_barrier()

    for k in range(SLICE // WB):
        pltpu.sync_copy(grid_s.at[pl.ds(sid * SLICE + k * WB, WB)],
                        bounce_v)
        pltpu.sync_copy(
            bounce_v,
            out_ref.at[pl.ds(OB_LEN + cid * GRID1 + sid * SLICE + k * WB,
                             WB)])

    @pl.when(jnp.logical_and(cid == 0, sid == 0))
    def _():
        pltpu.sync_copy(ob_in, ob_v)
        pltpu.sync_copy(ob_v, out_ref.at[pl.ds(0, OB_LEN)])


def kernel(ob, body_pos, body_depth, joint_posA, joint_posB, joint_depth,
           lidar_p2, hull):
    zx16 = jnp.full((16,), hull[0] - 0.5, jnp.float32)
    zy16 = jnp.full((16,), hull[1] - 0.5, jnp.float32)
    ones = jnp.ones((CHUNK,), jnp.float32)
    zeros = jnp.zeros((WB,), jnp.float32)
    out = _scatter_grid(
        body_pos.T.reshape(2 * NB), body_depth,
        joint_posA.T.reshape(2 * NJ), joint_posB.T.reshape(2 * NJ),
        joint_depth, lidar_p2.T.reshape(2 * NL),
        ob.reshape(OB_LEN), zeros, zx16, zy16, ones)
    return out.reshape(1, OUT_LEN)
